# baseline (speedup 1.0000x reference)
import functools

import jax
import jax.numpy as jnp
import numpy as np
from jax.experimental import pallas as pl
from jax.experimental.pallas import tpu as pltpu
from jax.experimental.pallas import tpu_sc as plsc

PMASK_SLOPE = 5.0
SAMPLE_SLOPE = 12.0
SPARSITY = 0.25


def _body(example_ref, weight_ref, thresh_ref, out_ref):
    p = jax.nn.sigmoid(weight_ref[...] * PMASK_SLOPE)
    pbar = jnp.mean(p)
    pmask = jnp.where(
        pbar > SPARSITY,
        p * (SPARSITY / pbar),
        1.0 - (1.0 - p) * ((1.0 - SPARSITY) / (1.0 - pbar)),
    )
    mask = jax.nn.sigmoid((pmask - thresh_ref[0]) * SAMPLE_SLOPE)
    out_ref[...] = example_ref[...] * mask[None, :, :]


@functools.partial(jax.jit, static_argnames=("row_block",))
def _loupe_mul(example3, weight2, thresh3, row_block):
    B, H, W = example3.shape
    grid = (B, H // row_block)
    return pl.pallas_call(
        _body,
        grid=grid,
        in_specs=[
            pl.BlockSpec((1, row_block, W), lambda b, r: (b, r, 0)),
            pl.BlockSpec((1, W), lambda b, r: (0, 0)),
            pl.BlockSpec((1, 1, W), lambda b, r: (b, 0, 0)),
        ],
        out_specs=pl.BlockSpec((1, row_block, W), lambda b, r: (b, r, 0)),
        out_shape=jax.ShapeDtypeStruct((B, H, W), example3.dtype),
        compiler_params=pltpu.CompilerParams(
            dimension_semantics=("parallel", "parallel"),
        ),
    )(example3, weight2, thresh3)


NROWS = 16384
WCOLS = 4096
NWORKERS = 32
ROWS_PER_WORKER = NROWS // NWORKERS
CHUNK_ROWS = 4
NCHUNK = ROWS_PER_WORKER // CHUNK_ROWS
NBUF = 4


def _sc_body(ex_hbm, w_hbm, t_hbm, out_hbm, wv, tv, mv,
             buf0, buf1, buf2, buf3,
             gsem0, gsem1, gsem2, gsem3, ssem0, ssem1, ssem2, ssem3):
    from jax import lax

    bufs = (buf0, buf1, buf2, buf3)
    gsems = (gsem0, gsem1, gsem2, gsem3)
    ssems = (ssem0, ssem1, ssem2, ssem3)

    cid = lax.axis_index("c")
    sid = lax.axis_index("s")
    wid = sid * 2 + cid
    base = wid * ROWS_PER_WORKER
    batch = wid // (NWORKERS // 4)

    pltpu.sync_copy(w_hbm, wv)
    pltpu.sync_copy(t_hbm.at[pl.ds(batch * WCOLS, WCOLS)], tv)

    nvec = WCOLS // 16

    def _sig(x):
        return 1.0 / (1.0 + jnp.exp(-x))

    def acc_body(i, acc):
        return acc + _sig(wv[pl.ds(i * 16, 16)] * PMASK_SLOPE)

    acc = lax.fori_loop(0, nvec, acc_body, jnp.zeros((16,), jnp.float32))
    sbar = acc[0]
    for j in range(1, 16):
        sbar = sbar + acc[j]
    xbar = sbar * (1.0 / WCOLS)

    xbar_v = lax.broadcast(xbar, (16,))
    scale_hi_v = SPARSITY / xbar_v
    scale_lo_v = (1.0 - SPARSITY) / (1.0 - xbar_v)
    ind = jnp.where(xbar > SPARSITY, 1.0, 0.0)
    coef_a = scale_hi_v * ind + scale_lo_v * (1.0 - ind)
    coef_c = (1.0 - scale_lo_v) * (1.0 - ind)

    def mask_body(i, carry):
        p = _sig(wv[pl.ds(i * 16, 16)] * PMASK_SLOPE)
        pm = p * coef_a + coef_c
        mv[pl.ds(i * 16, 16)] = _sig((pm - tv[pl.ds(i * 16, 16)]) * SAMPLE_SLOPE)
        return carry

    lax.fori_loop(0, nvec, mask_body, 0)

    def gather_start(g, b):
        return pltpu.async_copy(
            ex_hbm.at[pl.ds(base + g * CHUNK_ROWS, CHUNK_ROWS)], bufs[b],
            gsems[b])

    def gather_wait(g, b):
        pltpu.make_async_copy(
            ex_hbm.at[pl.ds(base + g * CHUNK_ROWS, CHUNK_ROWS)], bufs[b],
            gsems[b]).wait()

    def scatter_start(g, b):
        return pltpu.async_copy(
            bufs[b], out_hbm.at[pl.ds(base + g * CHUNK_ROWS, CHUNK_ROWS)],
            ssems[b])

    def scatter_wait(g, b):
        pltpu.make_async_copy(
            bufs[b], out_hbm.at[pl.ds(base + g * CHUNK_ROWS, CHUNK_ROWS)],
            ssems[b]).wait()

    def compute(b):
        buf = bufs[b]

        def cbody(i, carry):
            m = mv[pl.ds(i * 16, 16)]
            for r in range(CHUNK_ROWS):
                buf[r, pl.ds(i * 16, 16)] = buf[r, pl.ds(i * 16, 16)] * m
            return carry

        lax.fori_loop(0, nvec, cbody, 0)

    gather_start(0, 0)

    def step(t, carry):
        for b in range(NBUF):
            g = t * NBUF + b
            gather_wait(g, b)
            compute(b)
            scatter_start(g, b)
            nb = (b + 1) % NBUF

            @pl.when(g >= NBUF - 1)
            def _():
                scatter_wait(g - (NBUF - 1), nb)

            @pl.when(g + 1 < NCHUNK)
            def _():
                gather_start(g + 1, nb)
        return carry

    lax.fori_loop(0, NCHUNK // NBUF, step, 0)

    for b in range(1, NBUF):
        scatter_wait(NCHUNK - NBUF + b, b)


def _loupe_sc(example3, weight, thresh):
    B, H, W = example3.shape
    mesh = plsc.VectorSubcoreMesh(core_axis_name="c", subcore_axis_name="s")
    f = pl.kernel(
        _sc_body,
        out_type=jax.ShapeDtypeStruct((NROWS, WCOLS), jnp.float32),
        mesh=mesh,
        scratch_types=[
            pltpu.VMEM((WCOLS,), jnp.float32),
            pltpu.VMEM((WCOLS,), jnp.float32),
            pltpu.VMEM((WCOLS,), jnp.float32),
        ] + [pltpu.VMEM((CHUNK_ROWS, WCOLS), jnp.float32)] * NBUF
          + [pltpu.SemaphoreType.DMA] * (2 * NBUF),
    )
    return f(example3.reshape(NROWS, WCOLS), weight, thresh.reshape(-1))


def kernel(example, weight, thresh):
    B, C, H, W = example.shape
    ex3 = example.reshape(B, H, W)
    w2 = weight.reshape(1, W)
    t3 = thresh.reshape(B, 1, W)
    devs = jax.devices()
    ndev = 2 if (len(devs) >= 2 and B % 2 == 0) else 1
    if ndev == 1:
        out = _loupe_mul(ex3, w2, t3, row_block=512)
    else:
        mesh = jax.sharding.Mesh(np.asarray(devs[:ndev]), ("b",))
        f = jax.shard_map(
            lambda e, w, t: _loupe_mul(e, w, t, row_block=512),
            mesh=mesh,
            in_specs=(jax.sharding.PartitionSpec("b", None, None),
                      jax.sharding.PartitionSpec(None, None),
                      jax.sharding.PartitionSpec("b", None, None)),
            out_specs=jax.sharding.PartitionSpec("b", None, None),
            check_vma=False,
        )
        out = f(ex3, w2, t3)
    return out.reshape(B, C, H, W)


# speedup vs baseline: 3.8401x; 3.8401x over previous
import functools

import jax
import jax.numpy as jnp
from jax.experimental import pallas as pl
from jax.experimental.pallas import tpu as pltpu

PMASK_SLOPE = 5.0
SAMPLE_SLOPE = 12.0
SPARSITY = 0.25


def _body(example_ref, weight_ref, thresh_ref, out_ref):
    p = jax.nn.sigmoid(weight_ref[...] * PMASK_SLOPE)
    pbar = jnp.mean(p)
    pmask = jnp.where(
        pbar > SPARSITY,
        p * (SPARSITY / pbar),
        1.0 - (1.0 - p) * ((1.0 - SPARSITY) / (1.0 - pbar)),
    )
    mask = jax.nn.sigmoid((pmask - thresh_ref[0]) * SAMPLE_SLOPE)
    out_ref[...] = example_ref[...] * mask[None, :, :]


@functools.partial(jax.jit, static_argnames=("row_block",))
def _loupe_mul(example3, weight2, thresh3, row_block):
    B, H, W = example3.shape
    rpb = H // row_block
    grid = (B * rpb,)
    return pl.pallas_call(
        _body,
        grid=grid,
        in_specs=[
            pl.BlockSpec((1, row_block, W), lambda i: (i // rpb, i % rpb, 0)),
            pl.BlockSpec((1, W), lambda i: (0, 0)),
            pl.BlockSpec((1, 1, W), lambda i: (i // rpb, 0, 0)),
        ],
        out_specs=pl.BlockSpec((1, row_block, W),
                               lambda i: (i // rpb, i % rpb, 0)),
        out_shape=jax.ShapeDtypeStruct((B, H, W), example3.dtype),
        compiler_params=pltpu.CompilerParams(
            dimension_semantics=("arbitrary",),
        ),
    )(example3, weight2, thresh3)


def kernel(example, weight, thresh):
    B, C, H, W = example.shape
    out = _loupe_mul(example.reshape(B, H, W), weight.reshape(1, W),
                     thresh.reshape(B, 1, W), row_block=512)
    return out.reshape(B, C, H, W)


# speedup vs baseline: 3.8414x; 1.0003x over previous
import functools

import jax
import jax.numpy as jnp
from jax.experimental import pallas as pl
from jax.experimental.pallas import tpu as pltpu

PMASK_SLOPE = 5.0
SAMPLE_SLOPE = 12.0
SPARSITY = 0.25


def _body(example_ref, weight_ref, thresh_ref, out_ref):
    p = jax.nn.sigmoid(weight_ref[...] * PMASK_SLOPE)
    pbar = jnp.mean(p)
    pmask = jnp.where(
        pbar > SPARSITY,
        p * (SPARSITY / pbar),
        1.0 - (1.0 - p) * ((1.0 - SPARSITY) / (1.0 - pbar)),
    )
    mask = jax.nn.sigmoid((pmask - thresh_ref[0]) * SAMPLE_SLOPE)
    out_ref[...] = example_ref[...] * mask[None, :, :]


@functools.partial(jax.jit, static_argnames=("row_block",))
def _loupe_mul(example3, weight2, thresh3, row_block):
    B, H, W = example3.shape
    grid = (B, H // row_block)
    return pl.pallas_call(
        _body,
        grid=grid,
        in_specs=[
            pl.BlockSpec((1, row_block, W), lambda b, r: (b, r, 0)),
            pl.BlockSpec((1, W), lambda b, r: (0, 0)),
            pl.BlockSpec((1, 1, W), lambda b, r: (b, 0, 0)),
        ],
        out_specs=pl.BlockSpec((1, row_block, W), lambda b, r: (b, r, 0)),
        out_shape=jax.ShapeDtypeStruct((B, H, W), example3.dtype),
        compiler_params=pltpu.CompilerParams(
            dimension_semantics=("parallel", "parallel"),
        ),
    )(example3, weight2, thresh3)


def kernel(example, weight, thresh):
    B, C, H, W = example.shape
    out = _loupe_mul(example.reshape(B, H, W), weight.reshape(1, W),
                     thresh.reshape(B, 1, W), row_block=512)
    return out.reshape(B, C, H, W)


# speedup vs baseline: 3.9016x; 1.0157x over previous
import jax
import jax.numpy as jnp
from jax.experimental import pallas as pl
from jax.experimental.pallas import tpu as pltpu

PMASK_SLOPE = 5.0
SAMPLE_SLOPE = 12.0
SPARSITY = 0.25

CHUNK = 256
NBUF = 4


def _manual_body(w_ref, t_ref, ex_hbm, out_hbm, masks, *rest):
    from jax import lax

    nrows = ex_hbm.shape[0]
    nchunk = nrows // CHUNK
    chunks_per_batch = nchunk // t_ref.shape[0]
    bufs = rest[:NBUF]
    gsems = rest[NBUF:2 * NBUF]
    ssems = rest[2 * NBUF:]

    def gather_start(g, b):
        pltpu.make_async_copy(
            ex_hbm.at[pl.ds(g * CHUNK, CHUNK)], bufs[b], gsems[b]).start()

    def gather_wait(g, b):
        pltpu.make_async_copy(
            ex_hbm.at[pl.ds(g * CHUNK, CHUNK)], bufs[b], gsems[b]).wait()

    def scatter_start(g, b):
        pltpu.make_async_copy(
            bufs[b], out_hbm.at[pl.ds(g * CHUNK, CHUNK)], ssems[b]).start()

    def scatter_wait(g, b):
        pltpu.make_async_copy(
            bufs[b], out_hbm.at[pl.ds(g * CHUNK, CHUNK)], ssems[b]).wait()

    for g in range(NBUF - 1):
        gather_start(g, g)

    p = jax.nn.sigmoid(w_ref[...] * PMASK_SLOPE)
    pbar = jnp.mean(p)
    pmask = jnp.where(
        pbar > SPARSITY,
        p * (SPARSITY / pbar),
        1.0 - (1.0 - p) * ((1.0 - SPARSITY) / (1.0 - pbar)),
    )
    masks[...] = jax.nn.sigmoid((pmask - t_ref[...]) * SAMPLE_SLOPE)

    def step(t, carry):
        for b in range(NBUF):
            g = t * NBUF + b
            batch = g // chunks_per_batch
            gather_wait(g, b)
            m = masks[pl.ds(batch, 1), :]
            bufs[b][...] = bufs[b][...] * m
            scatter_start(g, b)
            nb = (b + NBUF - 1) % NBUF

            @pl.when(g >= 1)
            def _():
                scatter_wait(g - 1, nb)

            @pl.when(g + NBUF - 1 < nchunk)
            def _():
                gather_start(g + NBUF - 1, nb)
        return carry

    lax.fori_loop(0, nchunk // NBUF, step, 0, unroll=False)

    scatter_wait(nchunk - 1, (nchunk - 1) % NBUF)


@jax.jit
def _loupe_manual(example2, weight2, thresh2):
    R, W = example2.shape
    return pl.pallas_call(
        _manual_body,
        in_specs=[
            pl.BlockSpec(memory_space=pltpu.MemorySpace.VMEM),
            pl.BlockSpec(memory_space=pltpu.MemorySpace.VMEM),
            pl.BlockSpec(memory_space=pltpu.MemorySpace.HBM),
        ],
        out_specs=pl.BlockSpec(memory_space=pltpu.MemorySpace.HBM),
        out_shape=jax.ShapeDtypeStruct((R, W), example2.dtype),
        scratch_shapes=(
            [pltpu.VMEM(thresh2.shape, jnp.float32)]
            + [pltpu.VMEM((CHUNK, W), jnp.float32)] * NBUF
            + [pltpu.SemaphoreType.DMA] * (2 * NBUF)
        ),
    )(weight2, thresh2, example2)


def kernel(example, weight, thresh):
    B, C, H, W = example.shape
    out = _loupe_manual(example.reshape(B * H, W), weight.reshape(1, W),
                        thresh)
    return out.reshape(B, C, H, W)
